# Initial kernel scaffold; baseline (speedup 1.0000x reference)
import jax
import jax.numpy as jnp
from jax.experimental import pallas as pl


def kernel(toks, emb_weight):
    raise NotImplementedError("write your pallas kernel here")


# speedup vs baseline: 3.0512x; 3.0512x over previous
import functools

import jax
import jax.numpy as jnp
from jax import lax
from jax.experimental import pallas as pl
from jax.experimental.pallas import tpu as pltpu
from jax.experimental.pallas import tpu_sc as plsc

VOCAB_SIZE = 100000
D_MODEL = 128
BATCH = 16384
SEQ = 50

NC = 2
NS = 16
NW = NC * NS

TOTAL = BATCH * SEQ
PER_W = TOTAL // NW
CHUNK = 128
NCH = PER_W // CHUNK


def _gather_body(toks_hbm, table_hbm, out_hbm, idx_v, rows_v, sem):
    c = lax.axis_index("c")
    s = lax.axis_index("s")
    wid = s * NC + c

    pltpu.sync_copy(toks_hbm.at[wid], idx_v)

    def chunk_step(j, carry):
        pltpu.async_copy(table_hbm.at[idx_v.at[j]], rows_v, sem).wait()
        base = (wid * NCH + j) * CHUNK
        pltpu.sync_copy(rows_v, out_hbm.at[pl.ds(base, CHUNK)])
        return carry

    lax.fori_loop(0, NCH, chunk_step, 0)


@jax.jit
def _embed(toks_flat, emb_weight):
    mesh = plsc.VectorSubcoreMesh(core_axis_name="c", subcore_axis_name="s")
    k = functools.partial(
        pl.kernel,
        out_type=jax.ShapeDtypeStruct((TOTAL, D_MODEL), jnp.float32),
        mesh=mesh,
        scratch_types=[
            pltpu.VMEM((NCH, CHUNK), jnp.int32),
            pltpu.VMEM((CHUNK, D_MODEL), jnp.float32),
            pltpu.SemaphoreType.DMA,
        ],
    )(_gather_body)
    return k(toks_flat, emb_weight)


def kernel(toks, emb_weight):
    toks_flat = toks.astype(jnp.int32).reshape(NW, NCH, CHUNK)
    out = _embed(toks_flat, emb_weight)
    return out.reshape(BATCH, SEQ, D_MODEL)


# speedup vs baseline: 3.4459x; 1.1294x over previous
import functools

import jax
import jax.numpy as jnp
from jax import lax
from jax.experimental import pallas as pl
from jax.experimental.pallas import tpu as pltpu
from jax.experimental.pallas import tpu_sc as plsc

VOCAB_SIZE = 100000
D_MODEL = 128
BATCH = 16384
SEQ = 50

NC = 2
NS = 16
NW = NC * NS

TOTAL = BATCH * SEQ
PER_W = TOTAL // NW
CHUNK = 128
NCH = PER_W // CHUNK
NB = 4
NGRP = NCH // NB


def _gather_body(toks_hbm, table_hbm, out_hbm, idx_v, rows_v, gsem, wsem):
    c = lax.axis_index("c")
    s = lax.axis_index("s")
    wid = s * NC + c

    pltpu.sync_copy(toks_hbm.at[wid], idx_v)

    def wait_gather(b):
        pltpu.make_async_copy(
            table_hbm.at[pl.ds(0, CHUNK)], rows_v.at[b], gsem.at[b]
        ).wait()

    def wait_write(b):
        pltpu.make_async_copy(
            rows_v.at[b], out_hbm.at[pl.ds(0, CHUNK)], wsem.at[b]
        ).wait()

    def start_gather(b, j):
        pltpu.async_copy(table_hbm.at[idx_v.at[j]], rows_v.at[b], gsem.at[b])

    def start_write(b, j):
        base = (wid * NCH + j) * CHUNK
        pltpu.async_copy(rows_v.at[b], out_hbm.at[pl.ds(base, CHUNK)], wsem.at[b])

    for b in range(NB):
        start_gather(b, b)

    def group_step(g, carry):
        for b in range(NB):
            wait_gather(b)
            start_write(b, g * NB + b)
        for b in range(NB):
            wait_write(b)
            start_gather(b, (g + 1) * NB + b)
        return carry

    lax.fori_loop(0, NGRP - 1, group_step, 0)

    for b in range(NB):
        wait_gather(b)
        start_write(b, (NGRP - 1) * NB + b)
    for b in range(NB):
        wait_write(b)


@jax.jit
def _embed(toks_flat, emb_weight):
    mesh = plsc.VectorSubcoreMesh(core_axis_name="c", subcore_axis_name="s")
    k = functools.partial(
        pl.kernel,
        out_type=jax.ShapeDtypeStruct((TOTAL, D_MODEL), jnp.float32),
        mesh=mesh,
        scratch_types=[
            pltpu.VMEM((NCH, CHUNK), jnp.int32),
            pltpu.VMEM((NB, CHUNK, D_MODEL), jnp.float32),
            pltpu.SemaphoreType.DMA((NB,)),
            pltpu.SemaphoreType.DMA((NB,)),
        ],
    )(_gather_body)
    return k(toks_flat, emb_weight)


def kernel(toks, emb_weight):
    toks_flat = toks.astype(jnp.int32).reshape(NW, NCH, CHUNK)
    out = _embed(toks_flat, emb_weight)
    return out.reshape(BATCH, SEQ, D_MODEL)


# speedup vs baseline: 6.3583x; 1.8452x over previous
import functools

import jax
import jax.numpy as jnp
from jax import lax
from jax.experimental import pallas as pl
from jax.experimental.pallas import tpu as pltpu
from jax.experimental.pallas import tpu_sc as plsc

VOCAB_SIZE = 100000
D_MODEL = 128
BATCH = 16384
SEQ = 50

NC = 2
NS = 16
NW = NC * NS

ROWS_W = BATCH // NW
NB = 8
NGRP = ROWS_W // NB


def _gather_body(toks_hbm, table_hbm, out_hbm, idx_v, rows_v, gsem, wsem):
    c = lax.axis_index("c")
    s = lax.axis_index("s")
    wid = s * NC + c
    row0 = wid * ROWS_W

    pltpu.sync_copy(toks_hbm.at[wid], idx_v)

    def wait_gather(b, j):
        pltpu.make_async_copy(
            table_hbm.at[idx_v.at[j]], rows_v.at[b], gsem.at[b]
        ).wait()

    def wait_write(b):
        pltpu.make_async_copy(rows_v.at[b], out_hbm.at[0], wsem.at[b]).wait()

    def start_gather(b, j):
        pltpu.async_copy(table_hbm.at[idx_v.at[j]], rows_v.at[b], gsem.at[b])

    def start_write(b, j):
        pltpu.async_copy(rows_v.at[b], out_hbm.at[row0 + j], wsem.at[b])

    for b in range(NB):
        start_gather(b, b)

    def group_step(g, carry):
        for b in range(NB):
            j = g * NB + b
            wait_gather(b, j)
            start_write(b, j)
        for b in range(NB):
            wait_write(b)
            start_gather(b, (g + 1) * NB + b)
        return carry

    lax.fori_loop(0, NGRP - 1, group_step, 0)

    for b in range(NB):
        j = (NGRP - 1) * NB + b
        wait_gather(b, j)
        start_write(b, j)
    for b in range(NB):
        wait_write(b)


@jax.jit
def _embed(toks_chunked, emb_weight):
    mesh = plsc.VectorSubcoreMesh(core_axis_name="c", subcore_axis_name="s")
    k = functools.partial(
        pl.kernel,
        out_type=jax.ShapeDtypeStruct((BATCH, SEQ, D_MODEL), jnp.float32),
        mesh=mesh,
        scratch_types=[
            pltpu.VMEM((ROWS_W, SEQ), jnp.int32),
            pltpu.VMEM((NB, SEQ, D_MODEL), jnp.float32),
            pltpu.SemaphoreType.DMA((NB,)),
            pltpu.SemaphoreType.DMA((NB,)),
        ],
    )(_gather_body)
    return k(toks_chunked, emb_weight)


def kernel(toks, emb_weight):
    toks_chunked = toks.astype(jnp.int32).reshape(NW, ROWS_W, SEQ)
    return _embed(toks_chunked, emb_weight)
